# Initial kernel scaffold; baseline (speedup 1.0000x reference)
import jax
import jax.numpy as jnp
from jax.experimental import pallas as pl


def kernel(sampleU, sampleI, sampleR, alpha, betaU, betaI):
    raise NotImplementedError("write your pallas kernel here")


# speedup vs baseline: 1.2484x; 1.2484x over previous
import functools

import jax
import jax.numpy as jnp
from jax import lax
from jax.experimental import pallas as pl
from jax.experimental.pallas import tpu as pltpu
from jax.experimental.pallas import tpu_sc as plsc

_NC = 2
_NS = 16
_NW = _NC * _NS
_B = 16384
_BPW = _B // _NW
_KC = 4
_CW = _BPW // _KC
_L = 16
_VPC = _CW // _L


def _make_sc_kernel():
    mesh = plsc.VectorSubcoreMesh(core_axis_name="c", subcore_axis_name="s")

    @functools.partial(
        pl.kernel,
        mesh=mesh,
        out_type=jax.ShapeDtypeStruct((_NW, _L), jnp.float32),
        scratch_types=[
            pltpu.VMEM((_KC, _CW), jnp.int32),
            pltpu.VMEM((_KC, _CW), jnp.int32),
            pltpu.VMEM((_KC, _CW), jnp.float32),
            pltpu.VMEM((_KC, _CW), jnp.float32),
            pltpu.VMEM((_KC, _CW), jnp.float32),
            pltpu.VMEM((_L,), jnp.float32),
            pltpu.VMEM((_L,), jnp.float32),
            pltpu.SemaphoreType.DMA,
        ],
    )
    def _k(su_hbm, si_hbm, r_hbm, alpha_hbm, bu_hbm, bi_hbm, out_hbm,
           idxu_v, idxi_v, bu_v, bi_v, r_v, alpha_v, acc_v, sem):
        cid = lax.axis_index("c")
        sid = lax.axis_index("s")
        wid = sid * _NC + cid

        pltpu.sync_copy(su_hbm.at[wid], idxu_v)
        pltpu.sync_copy(si_hbm.at[wid], idxi_v)
        pltpu.sync_copy(r_hbm.at[wid], r_v)
        pltpu.sync_copy(alpha_hbm, alpha_v)

        copies = []
        for k in range(_KC):
            copies.append(
                pltpu.async_copy(bu_hbm.at[idxu_v.at[k]], bu_v.at[k], sem))
            copies.append(
                pltpu.async_copy(bi_hbm.at[idxi_v.at[k]], bi_v.at[k], sem))
        for c in copies:
            c.wait()

        av = alpha_v[...]
        acc = jnp.zeros((_L,), jnp.float32)
        for k in range(_KC):
            for j in range(_VPC):
                sl = pl.ds(j * _L, _L)
                d = av + bu_v[k, sl] + bi_v[k, sl] - r_v[k, sl]
                acc = acc + d * d
        acc_v[...] = acc * (0.5 / _B)
        pltpu.sync_copy(acc_v, out_hbm.at[wid])

    return _k


_sc_kernel = _make_sc_kernel()


def kernel(sampleU, sampleI, sampleR, alpha, betaU, betaI):
    su = sampleU.astype(jnp.int32).reshape(_NW, _KC, _CW)
    si = sampleI.astype(jnp.int32).reshape(_NW, _KC, _CW)
    r = sampleR.astype(jnp.float32).reshape(_NW, _KC, _CW)
    al = jnp.broadcast_to(alpha.astype(jnp.float32), (_L,))
    partials = _sc_kernel(su, si, r, al, betaU, betaI)
    return jnp.sum(partials)


# speedup vs baseline: 1.3395x; 1.0730x over previous
import functools

import jax
import jax.numpy as jnp
from jax import lax
from jax.experimental import pallas as pl
from jax.experimental.pallas import tpu as pltpu
from jax.experimental.pallas import tpu_sc as plsc

_NC = 2
_NS = 16
_NW = _NC * _NS
_B = 16384
_BPW = _B // _NW
_KC = 4
_CW = _BPW // _KC
_L = 16
_VPC = _CW // _L


def _make_sc_kernel():
    mesh = plsc.VectorSubcoreMesh(core_axis_name="c", subcore_axis_name="s")

    @functools.partial(
        pl.kernel,
        mesh=mesh,
        out_type=jax.ShapeDtypeStruct((_NW, _L), jnp.float32),
        scratch_types=[
            pltpu.VMEM((_KC, _CW), jnp.int32),
            pltpu.VMEM((_KC, _CW), jnp.int32),
            pltpu.VMEM((_KC, _CW), jnp.float32),
            pltpu.VMEM((_KC, _CW), jnp.float32),
            pltpu.VMEM((_KC, _CW), jnp.float32),
            pltpu.VMEM((_L,), jnp.float32),
            pltpu.VMEM((_L,), jnp.float32),
            pltpu.SemaphoreType.DMA,
            pltpu.SemaphoreType.DMA,
        ],
    )
    def _k(su_hbm, si_hbm, r_hbm, alpha_hbm, bu_hbm, bi_hbm, out_hbm,
           idxu_v, idxi_v, bu_v, bi_v, r_v, alpha_v, acc_v, sem_in, sem_g):
        cid = lax.axis_index("c")
        sid = lax.axis_index("s")
        wid = sid * _NC + cid

        ld_u = pltpu.async_copy(su_hbm.at[wid], idxu_v, sem_in)
        ld_i = pltpu.async_copy(si_hbm.at[wid], idxi_v, sem_in)
        ld_r = pltpu.async_copy(r_hbm.at[wid], r_v, sem_in)
        ld_a = pltpu.async_copy(alpha_hbm, alpha_v, sem_in)
        ld_u.wait()
        ld_i.wait()
        copies = []
        for k in range(_KC):
            copies.append(
                pltpu.async_copy(bu_hbm.at[idxu_v.at[k]], bu_v.at[k], sem_g))
            copies.append(
                pltpu.async_copy(bi_hbm.at[idxi_v.at[k]], bi_v.at[k], sem_g))
        ld_r.wait()
        ld_a.wait()
        for c in copies:
            c.wait()

        av = alpha_v[...]
        acc = jnp.zeros((_L,), jnp.float32)
        for k in range(_KC):
            for j in range(_VPC):
                sl = pl.ds(j * _L, _L)
                d = av + bu_v[k, sl] + bi_v[k, sl] - r_v[k, sl]
                acc = acc + d * d
        acc_v[...] = acc * (0.5 / _B)
        pltpu.sync_copy(acc_v, out_hbm.at[wid])

    return _k


_sc_kernel = _make_sc_kernel()


def kernel(sampleU, sampleI, sampleR, alpha, betaU, betaI):
    su = sampleU.astype(jnp.int32).reshape(_NW, _KC, _CW)
    si = sampleI.astype(jnp.int32).reshape(_NW, _KC, _CW)
    r = sampleR.astype(jnp.float32).reshape(_NW, _KC, _CW)
    al = jnp.broadcast_to(alpha.astype(jnp.float32), (_L,))
    partials = _sc_kernel(su, si, r, al, betaU, betaI)
    return jnp.sum(partials)
